# baseline (speedup 1.0000x reference)
import functools

import numpy as np
import jax
import jax.numpy as jnp
from jax import lax
from jax.experimental import pallas as pl
from jax.experimental.pallas import tpu as pltpu
from jax.experimental.pallas import tpu_sc as plsc

_L = 16
_NW = 16


@functools.lru_cache(maxsize=None)
def _make_sc_kernel(B, C, D, H, W, K):
    n = B * K * C
    per_w = n // (2 * _NW)
    nj = per_w // _L
    k_per_w = per_w // C
    w_per_b = 2 * _NW // B
    blk = 8 * W
    blocks_per_bc = D * H // 8
    denom = np.float32(8.0 * B + 0.0001)

    mesh = plsc.VectorSubcoreMesh(
        core_axis_name="c", subcore_axis_name="s", num_cores=2)

    @functools.partial(
        pl.kernel,
        mesh=mesh,
        compiler_params=pltpu.CompilerParams(needs_layout_passes=False),
        out_type=jax.ShapeDtypeStruct((2 * _L,), jnp.float32),
        scratch_types=[
            pltpu.VMEM((k_per_w,), jnp.int32),
            pltpu.VMEM((per_w * 8,), jnp.float32),
            pltpu.VMEM((per_w,), jnp.float32),
            pltpu.VMEM((_L,), jnp.float32),
            pltpu.VMEM((_NW * _L,), jnp.float32),
            pltpu.VMEM_SHARED((_NW * _L,), jnp.float32),
            pltpu.SemaphoreType.DMA,
        ],
    )
    def sc_kernel(outp, ind, tgt, out, ind_v, lin_v, tgt_v,
                  stage_v, red_v, shared, sem):
        cid = lax.axis_index("c")
        sid = lax.axis_index("s")

        def _worker():
            wid = cid * _NW + sid
            base_e = wid * per_w
            base_k = wid * k_per_w
            pltpu.sync_copy(ind.at[pl.ds(base_k, k_per_w)], ind_v)

            bc_base = jnp.full((_L,), (wid // w_per_b) * C, jnp.int32)

            lane = lax.iota(jnp.int32, _L)
            c_vec = jnp.full((_L,), C, jnp.int32)
            w_vec = jnp.full((_L,), W, jnp.int32)
            blk_vec = jnp.full((_L,), blk, jnp.int32)
            e8_vec = jnp.full((_L,), 8, jnp.int32)
            qv, ivs = [], []
            for j in range(nj):
                lanes = lane + (j * _L)
                f_loc = lax.div(lanes, c_vec)
                c16 = lax.rem(lanes, c_vec)
                iv16 = plsc.load_gather(ind_v, [f_loc])
                dh = lax.div(iv16, w_vec)
                w16 = iv16 - dh * W
                g16 = (bc_base + c16) * blocks_per_bc + lax.div(iv16, blk_vec)
                s16 = dh - lax.div(iv16, blk_vec) * 8
                q16 = g16 * 1024 + s16 * 128 + (w16 - lax.rem(w16, e8_vec))
                qv.append(q16)
                ivs.append(iv16)

            for e in range(per_w):
                j, l = e // _L, e % _L
                q = qv[j][l]
                g = lax.shift_right_logical(q, 10)
                s = lax.bitwise_and(lax.shift_right_logical(q, 7), 7)
                w8 = pl.multiple_of(lax.bitwise_and(q, 127), 8)
                pltpu.async_copy(outp.at[g, s, pl.ds(w8, 8)],
                                 lin_v.at[pl.ds(e * 8, 8)], sem)

            pltpu.sync_copy(tgt.at[pl.ds(base_e, per_w)], tgt_v)
            pltpu.make_async_copy(
                tgt.at[pl.ds(0, per_w * 8)], lin_v, sem).wait()

            acc = jnp.zeros((_L,), jnp.float32)
            for j in range(nj):
                lanes = lane + (j * _L)
                idx = lanes * 8 + lax.rem(ivs[j], e8_vec)
                v = plsc.load_gather(lin_v, [idx])
                t = tgt_v[pl.ds(j * _L, _L)]
                acc = acc + jnp.abs(v - t)

            stage_v[...] = acc
            pltpu.sync_copy(stage_v, shared.at[pl.ds(sid * _L, _L)])
            plsc.subcore_barrier()

            @pl.when(sid == 0)
            def _reduce():
                pltpu.sync_copy(shared, red_v)
                tot = jnp.zeros((_L,), jnp.float32)
                for i in range(_NW):
                    tot = tot + red_v[pl.ds(i * _L, _L)]
                dnums = lax.GatherDimensionNumbers(
                    offset_dims=(), collapsed_slice_dims=(0,),
                    start_index_map=(0,))
                for sh in (8, 4, 2, 1):
                    perm = lax.iota(jnp.int32, _L) ^ sh
                    tot = tot + lax.gather(
                        tot, perm[:, None], dimension_numbers=dnums,
                        slice_sizes=(1,),
                        mode=lax.GatherScatterMode.PROMISE_IN_BOUNDS)
                stage_v[...] = tot / denom
                pltpu.sync_copy(stage_v, out.at[pl.ds(cid * _L, _L)])

        _worker()

    return sc_kernel


def kernel(output, ind, target):
    B, C, D, H, W = output.shape
    K = ind.shape[1]
    outp3d = output.reshape(B * C * D * H // 8, 8, W)
    ind_flat = ind.reshape(-1).astype(jnp.int32)
    tgt = target.reshape(-1)
    res = _make_sc_kernel(B, C, D, H, W, K)(outp3d, ind_flat, tgt)
    return res[0] + res[_L]


# speedup vs baseline: 1.1195x; 1.1195x over previous
import functools

import numpy as np
import jax
import jax.numpy as jnp
from jax import lax
from jax.experimental import pallas as pl
from jax.experimental.pallas import tpu as pltpu
from jax.experimental.pallas import tpu_sc as plsc

_L = 16
_NW = 16


@functools.lru_cache(maxsize=None)
def _make_sc_kernel(B, C, D, H, W, K):
    n = B * K * C
    per_w = n // _NW
    nj = per_w // _L
    k_per_w = per_w // C
    half = per_w // 2
    nrow = half // _L
    w_per_b = _NW // B
    blk = 8 * W
    blocks_per_bc = D * H // 8
    denom = np.float32(8.0 * B + 0.0001)

    mesh = plsc.VectorSubcoreMesh(
        core_axis_name="c", subcore_axis_name="s", num_cores=1)

    @functools.partial(
        pl.kernel,
        mesh=mesh,
        compiler_params=pltpu.CompilerParams(needs_layout_passes=False),
        out_type=jax.ShapeDtypeStruct((_L,), jnp.float32),
        scratch_types=[
            pltpu.VMEM((k_per_w,), jnp.int32),
            pltpu.VMEM((per_w * 8,), jnp.float32),
            pltpu.VMEM((per_w,), jnp.float32),
            pltpu.VMEM((_L,), jnp.float32),
            pltpu.VMEM((_NW * _L,), jnp.float32),
            pltpu.VMEM_SHARED((_NW * _L,), jnp.float32),
            pltpu.SemaphoreType.DMA,
        ],
    )
    def sc_kernel(outp, ind, tgt, out, ind_v, lin_v, tgt_v,
                  stage_v, red_v, shared, sem):
        cid = lax.axis_index("c")
        sid = lax.axis_index("s")

        @pl.when(cid == 0)
        def _core0():
            base_e = sid * per_w
            base_k = sid * k_per_w
            pltpu.sync_copy(ind.at[pl.ds(base_k, k_per_w)], ind_v)

            bc_base = jnp.full((_L,), (sid // w_per_b) * C, jnp.int32)

            lane = lax.iota(jnp.int32, _L)
            c_vec = jnp.full((_L,), C, jnp.int32)
            w_vec = jnp.full((_L,), W, jnp.int32)
            blk_vec = jnp.full((_L,), blk, jnp.int32)
            e8_vec = jnp.full((_L,), 8, jnp.int32)
            qv, ivs = [], []
            for j in range(nj):
                lanes = lane + (j * _L)
                f_loc = lax.div(lanes, c_vec)
                c16 = lax.rem(lanes, c_vec)
                iv16 = plsc.load_gather(ind_v, [f_loc])
                dh = lax.div(iv16, w_vec)
                w16 = iv16 - dh * W
                g16 = (bc_base + c16) * blocks_per_bc + lax.div(iv16, blk_vec)
                s16 = dh - lax.div(iv16, blk_vec) * 8
                q16 = g16 * 1024 + s16 * 128 + (w16 - lax.rem(w16, e8_vec))
                qv.append(q16)
                ivs.append(iv16)

            for e in range(per_w):
                j, l = e // _L, e % _L
                q = qv[j][l]
                g = lax.shift_right_logical(q, 10)
                s = lax.bitwise_and(lax.shift_right_logical(q, 7), 7)
                w8 = pl.multiple_of(lax.bitwise_and(q, 127), 8)
                pltpu.async_copy(outp.at[g, s, pl.ds(w8, 8)],
                                 lin_v.at[pl.ds(e * 8, 8)], sem)

            pltpu.sync_copy(tgt.at[pl.ds(base_e, per_w)], tgt_v)
            pltpu.make_async_copy(
                tgt.at[pl.ds(0, per_w * 8)], lin_v, sem).wait()

            acc = jnp.zeros((_L,), jnp.float32)
            for j in range(nj):
                lanes = lane + (j * _L)
                idx = lanes * 8 + lax.rem(ivs[j], e8_vec)
                v = plsc.load_gather(lin_v, [idx])
                t = tgt_v[pl.ds(j * _L, _L)]
                acc = acc + jnp.abs(v - t)

            stage_v[...] = acc
            pltpu.sync_copy(stage_v, shared.at[pl.ds(sid * _L, _L)])
            plsc.subcore_barrier()

            @pl.when(sid == 0)
            def _reduce():
                pltpu.sync_copy(shared, red_v)
                tot = jnp.zeros((_L,), jnp.float32)
                for i in range(_NW):
                    tot = tot + red_v[pl.ds(i * _L, _L)]
                dnums = lax.GatherDimensionNumbers(
                    offset_dims=(), collapsed_slice_dims=(0,),
                    start_index_map=(0,))
                for sh in (8, 4, 2, 1):
                    perm = lax.iota(jnp.int32, _L) ^ sh
                    tot = tot + lax.gather(
                        tot, perm[:, None], dimension_numbers=dnums,
                        slice_sizes=(1,),
                        mode=lax.GatherScatterMode.PROMISE_IN_BOUNDS)
                stage_v[...] = tot / denom
                pltpu.sync_copy(stage_v, out)

    return sc_kernel


def kernel(output, ind, target):
    B, C, D, H, W = output.shape
    K = ind.shape[1]
    outp3d = output.reshape(B * C * D * H // 8, 8, W)
    ind_flat = ind.reshape(-1).astype(jnp.int32)
    tgt = target.reshape(-1)
    res = _make_sc_kernel(B, C, D, H, W, K)(outp3d, ind_flat, tgt)
    return res[0]


# speedup vs baseline: 1.1443x; 1.0222x over previous
import functools

import numpy as np
import jax
import jax.numpy as jnp
from jax import lax
from jax.experimental import pallas as pl
from jax.experimental.pallas import tpu as pltpu
from jax.experimental.pallas import tpu_sc as plsc

_L = 16
_NW = 16


@functools.lru_cache(maxsize=None)
def _make_sc_kernel(B, C, D, H, W, K):
    n = B * K * C
    per_w = n // _NW
    nj = per_w // _L
    k_per_w = per_w // C
    half = per_w // 2
    nrow = half // _L
    w_per_b = _NW // B
    blk = 8 * W
    blocks_per_bc = D * H // 8
    denom = np.float32(8.0 * B + 0.0001)

    mesh = plsc.VectorSubcoreMesh(
        core_axis_name="c", subcore_axis_name="s", num_cores=1)

    @functools.partial(
        pl.kernel,
        mesh=mesh,
        compiler_params=pltpu.CompilerParams(needs_layout_passes=False),
        out_type=jax.ShapeDtypeStruct((_L,), jnp.float32),
        scratch_types=[
            pltpu.VMEM((k_per_w,), jnp.int32),
            pltpu.VMEM((per_w * 8,), jnp.float32),
            pltpu.VMEM((per_w,), jnp.float32),
            pltpu.VMEM((_L,), jnp.float32),
            pltpu.VMEM((_NW * _L,), jnp.float32),
            pltpu.VMEM_SHARED((_NW * _L,), jnp.float32),
            pltpu.SemaphoreType.DMA,
        ],
    )
    def sc_kernel(outp, ind, tgt, out, ind_v, lin_v, tgt_v,
                  stage_v, red_v, shared, sem):
        cid = lax.axis_index("c")
        sid = lax.axis_index("s")

        @pl.when(cid == 0)
        def _core0():
            base_e = sid * per_w
            base_k = sid * k_per_w
            pltpu.sync_copy(ind.at[pl.ds(base_k, k_per_w)], ind_v)

            bc_base = jnp.full((_L,), (sid // w_per_b) * C, jnp.int32)

            lane = lax.iota(jnp.int32, _L)
            c_vec = jnp.full((_L,), C, jnp.int32)
            w_vec = jnp.full((_L,), W, jnp.int32)
            blk_vec = jnp.full((_L,), blk, jnp.int32)
            e8_vec = jnp.full((_L,), 8, jnp.int32)
            qv, ivs = [], []
            for j in range(nj):
                lanes = lane + (j * _L)
                f_loc = lax.div(lanes, c_vec)
                c16 = lax.rem(lanes, c_vec)
                iv16 = plsc.load_gather(ind_v, [f_loc])
                dh = lax.div(iv16, w_vec)
                w16 = iv16 - dh * W
                r16 = (bc_base + c16) * (D * H) + dh
                q16 = r16 * 128 + (w16 - lax.rem(w16, e8_vec))
                qv.append(q16)
                ivs.append(iv16)

            for e in range(per_w):
                j, l = e // _L, e % _L
                q = qv[j][l]
                r = lax.shift_right_logical(q, 7)
                w8 = pl.multiple_of(lax.bitwise_and(q, 127), 8)
                pltpu.async_copy(outp.at[r, pl.ds(w8, 8)],
                                 lin_v.at[pl.ds(e * 8, 8)], sem)

            pltpu.sync_copy(tgt.at[pl.ds(base_e, per_w)], tgt_v)
            pltpu.make_async_copy(
                tgt.at[pl.ds(0, per_w * 8)], lin_v, sem).wait()

            acc = jnp.zeros((_L,), jnp.float32)
            for j in range(nj):
                lanes = lane + (j * _L)
                idx = lanes * 8 + lax.rem(ivs[j], e8_vec)
                v = plsc.load_gather(lin_v, [idx])
                t = tgt_v[pl.ds(j * _L, _L)]
                acc = acc + jnp.abs(v - t)

            stage_v[...] = acc
            pltpu.sync_copy(stage_v, shared.at[pl.ds(sid * _L, _L)])
            plsc.subcore_barrier()

            @pl.when(sid == 0)
            def _reduce():
                pltpu.sync_copy(shared, red_v)
                tot = jnp.zeros((_L,), jnp.float32)
                for i in range(_NW):
                    tot = tot + red_v[pl.ds(i * _L, _L)]
                dnums = lax.GatherDimensionNumbers(
                    offset_dims=(), collapsed_slice_dims=(0,),
                    start_index_map=(0,))
                for sh in (8, 4, 2, 1):
                    perm = lax.iota(jnp.int32, _L) ^ sh
                    tot = tot + lax.gather(
                        tot, perm[:, None], dimension_numbers=dnums,
                        slice_sizes=(1,),
                        mode=lax.GatherScatterMode.PROMISE_IN_BOUNDS)
                stage_v[...] = tot / denom
                pltpu.sync_copy(stage_v, out)

    return sc_kernel


def kernel(output, ind, target):
    B, C, D, H, W = output.shape
    K = ind.shape[1]
    outp3d = output.reshape(B * C * D * H, W)
    ind_flat = ind.reshape(-1).astype(jnp.int32)
    tgt = target.reshape(-1)
    res = _make_sc_kernel(B, C, D, H, W, K)(outp3d, ind_flat, tgt)
    return res[0]


# speedup vs baseline: 1.1453x; 1.0008x over previous
import functools

import numpy as np
import jax
import jax.numpy as jnp
from jax import lax
from jax.experimental import pallas as pl
from jax.experimental.pallas import tpu as pltpu
from jax.experimental.pallas import tpu_sc as plsc

_L = 16
_NW = 16


@functools.lru_cache(maxsize=None)
def _make_sc_kernel(B, C, D, H, W, K):
    n = B * K * C
    per_w = n // _NW
    nj = per_w // _L
    k_per_w = per_w // C
    w_per_b = _NW // B
    denom = np.float32(8.0 * B + 0.0001)

    mesh = plsc.VectorSubcoreMesh(
        core_axis_name="c", subcore_axis_name="s", num_cores=1)

    @functools.partial(
        pl.kernel,
        mesh=mesh,
        compiler_params=pltpu.CompilerParams(needs_layout_passes=False),
        out_type=jax.ShapeDtypeStruct((_L,), jnp.float32),
        scratch_types=[
            pltpu.VMEM((k_per_w,), jnp.int32),
            pltpu.VMEM((per_w * 8,), jnp.float32),
            pltpu.VMEM((per_w,), jnp.float32),
            pltpu.VMEM((_L,), jnp.float32),
            pltpu.VMEM((_NW * _L,), jnp.float32),
            pltpu.VMEM_SHARED((_NW * _L,), jnp.float32),
            pltpu.SemaphoreType.DMA,
        ],
    )
    def sc_kernel(outp, ind, tgt, out, ind_v, lin_v, tgt_v,
                  stage_v, red_v, shared, sem):
        cid = lax.axis_index("c")
        sid = lax.axis_index("s")

        @pl.when(cid == 0)
        def _core0():
            base_e = sid * per_w
            base_k = sid * k_per_w
            pltpu.sync_copy(ind.at[pl.ds(base_k, k_per_w)], ind_v)

            bc_base = jnp.full((_L,), (sid // w_per_b) * C, jnp.int32)

            lane = lax.iota(jnp.int32, _L)
            c_vec = jnp.full((_L,), C, jnp.int32)
            w_vec = jnp.full((_L,), W, jnp.int32)
            e8_vec = jnp.full((_L,), 8, jnp.int32)
            qv, ivs = [], []
            for j in range(nj):
                lanes = lane + (j * _L)
                f_loc = lax.div(lanes, c_vec)
                c16 = lax.rem(lanes, c_vec)
                iv16 = plsc.load_gather(ind_v, [f_loc])
                dh = lax.div(iv16, w_vec)
                w16 = iv16 - dh * W
                r16 = (bc_base + c16) * (D * H) + dh
                q16 = r16 * 128 + (w16 - lax.rem(w16, e8_vec))
                qv.append(q16)
                ivs.append(iv16)

            for e in range(per_w):
                j, l = e // _L, e % _L
                q = qv[j][l]
                r = lax.shift_right_logical(q, 7)
                w8 = pl.multiple_of(lax.bitwise_and(q, 127), 8)
                pltpu.async_copy(outp.at[r, pl.ds(w8, 8)],
                                 lin_v.at[pl.ds(e * 8, 8)], sem)

            pltpu.sync_copy(tgt.at[pl.ds(base_e, per_w)], tgt_v)
            pltpu.make_async_copy(
                tgt.at[pl.ds(0, per_w * 8)], lin_v, sem).wait()

            acc = jnp.zeros((_L,), jnp.float32)
            for j in range(nj):
                lanes = lane + (j * _L)
                idx = lanes * 8 + lax.rem(ivs[j], e8_vec)
                v = plsc.load_gather(lin_v, [idx])
                t = tgt_v[pl.ds(j * _L, _L)]
                acc = acc + jnp.abs(v - t)

            stage_v[...] = acc
            pltpu.sync_copy(stage_v, shared.at[pl.ds(sid * _L, _L)])
            plsc.subcore_barrier()

            @pl.when(sid == 0)
            def _reduce():
                pltpu.sync_copy(shared, red_v)
                tot = jnp.zeros((_L,), jnp.float32)
                for i in range(_NW):
                    tot = tot + red_v[pl.ds(i * _L, _L)]
                dnums = lax.GatherDimensionNumbers(
                    offset_dims=(), collapsed_slice_dims=(0,),
                    start_index_map=(0,))
                for sh in (8, 4, 2, 1):
                    perm = lax.iota(jnp.int32, _L) ^ sh
                    tot = tot + lax.gather(
                        tot, perm[:, None], dimension_numbers=dnums,
                        slice_sizes=(1,),
                        mode=lax.GatherScatterMode.PROMISE_IN_BOUNDS)
                stage_v[...] = tot / denom
                pltpu.sync_copy(stage_v, out)

    return sc_kernel


def kernel(output, ind, target):
    B, C, D, H, W = output.shape
    K = ind.shape[1]
    outp2d = output.reshape(B * C * D * H, W)
    ind_flat = ind.reshape(-1).astype(jnp.int32)
    tgt = target.reshape(-1)
    res = _make_sc_kernel(B, C, D, H, W, K)(outp2d, ind_flat, tgt)
    return res[0]
